# initial kernel scaffold (unmeasured)
import jax
import jax.numpy as jnp
from jax import lax
from jax.experimental import pallas as pl
from jax.experimental.pallas import tpu as pltpu


def kernel(
    x,
):
    def body(*refs):
        pass

    out_shape = jax.ShapeDtypeStruct(..., jnp.float32)
    return pl.pallas_call(body, out_shape=out_shape)(...)


# baseline (device time: 54291 ns/iter reference)
import jax
import jax.numpy as jnp
from jax import lax
from jax.experimental import pallas as pl
from jax.experimental.pallas import tpu as pltpu

N_Y = 4


def kernel(x):
    m_per, n_full = x.shape
    n_per = n_full // N_Y
    m_full = m_per * N_Y

    def body(x_ref, out_ref, xb_ref, send_sems, recv_sems):
        my_x = lax.axis_index("x")
        my_y = lax.axis_index("y")
        my_z = lax.axis_index("z")

        xb_ref[...] = x_ref[...].astype(jnp.bfloat16)

        barrier_sem = pltpu.get_barrier_semaphore()
        for d in range(1, N_Y):
            peer = (my_y + d) % N_Y
            pl.semaphore_signal(
                barrier_sem, inc=1,
                device_id=(my_x, peer, my_z),
                device_id_type=pl.DeviceIdType.MESH,
            )
        pl.semaphore_wait(barrier_sem, N_Y - 1)

        out_ref[pl.ds(my_y * m_per, m_per), :] = (
            xb_ref[:, pl.ds(my_y * n_per, n_per)]
        )

        sends = []
        for d in range(1, N_Y):
            dst_y = (my_y + d) % N_Y
            rdma = pltpu.make_async_remote_copy(
                src_ref=xb_ref.at[:, pl.ds(dst_y * n_per, n_per)],
                dst_ref=out_ref.at[pl.ds(my_y * m_per, m_per), :],
                send_sem=send_sems.at[d - 1],
                recv_sem=recv_sems.at[d - 1],
                device_id=(my_x, dst_y, my_z),
                device_id_type=pl.DeviceIdType.MESH,
            )
            rdma.start()
            sends.append(rdma)

        for d in range(1, N_Y):
            src_y = (my_y - d) % N_Y
            recv = pltpu.make_async_remote_copy(
                src_ref=xb_ref.at[:, pl.ds(src_y * n_per, n_per)],
                dst_ref=out_ref.at[pl.ds(src_y * m_per, m_per), :],
                send_sem=send_sems.at[d - 1],
                recv_sem=recv_sems.at[d - 1],
                device_id=(my_x, src_y, my_z),
                device_id_type=pl.DeviceIdType.MESH,
            )
            recv.wait_recv()

        for rdma in sends:
            rdma.wait_send()

    return pl.pallas_call(
        body,
        out_shape=jax.ShapeDtypeStruct((m_full, n_per), jnp.bfloat16),
        in_specs=[pl.BlockSpec(memory_space=pltpu.VMEM)],
        out_specs=pl.BlockSpec(memory_space=pltpu.VMEM),
        scratch_shapes=[
            pltpu.VMEM((m_per, n_full), jnp.bfloat16),
            pltpu.SemaphoreType.DMA((N_Y - 1,)),
            pltpu.SemaphoreType.DMA((N_Y - 1,)),
        ],
        compiler_params=pltpu.CompilerParams(collective_id=0),
    )(x)
